# Initial kernel scaffold; baseline (speedup 1.0000x reference)
import jax
import jax.numpy as jnp
from jax.experimental import pallas as pl


def kernel(x, edge_index, batch, eps0, W1_0, b1_0, W2_0, b2_0, eps1, W1_1, b1_1, W2_1, b2_1, Wc, bc):
    raise NotImplementedError("write your pallas kernel here")


# speedup vs baseline: 8.2900x; 8.2900x over previous
import functools

import jax
import jax.numpy as jnp
from jax import lax
from jax.experimental import pallas as pl
from jax.experimental.pallas import tpu as pltpu
from jax.experimental.pallas import tpu_sc as plsc

N = 10000
E = 320000
D = 128
H = 128
C = 16
G = 64

NW = 32
EPW = E // NW
CH = 40
CHH = 125
NP = 10240
RPS = NP // 16


def _segment_sum_sc(h, src_r, dst_r, zrows):
    mesh = plsc.VectorSubcoreMesh(core_axis_name="c", subcore_axis_name="s")

    @functools.partial(
        pl.kernel,
        mesh=mesh,
        out_type=jax.ShapeDtypeStruct((2, NP, D), jnp.float32),
        scratch_types=[
            pltpu.VMEM((CHH, CH), jnp.int32),
            pltpu.VMEM((CHH, CH), jnp.int32),
            pltpu.VMEM((CH, D), jnp.float32),
            pltpu.VMEM((CH, D), jnp.float32),
            pltpu.VMEM_SHARED((NP, D), jnp.float32),
            pltpu.SemaphoreType.DMA,
            pltpu.SemaphoreType.DMA,
        ],
    )
    def agg(h_hbm, src_hbm, dst_hbm, z_hbm, out_hbm,
            src_v, dst_v, buf0, buf1, acc, sem0, sem1):
        cid = lax.axis_index("c")
        sid = lax.axis_index("s")
        wid = sid * 2 + cid

        pltpu.sync_copy(z_hbm, acc.at[pl.ds(sid * RPS, RPS)])
        plsc.subcore_barrier()

        for half in range(2):
            pltpu.sync_copy(src_hbm.at[wid, half], src_v)
            pltpu.sync_copy(dst_hbm.at[wid, half], dst_v)
            pltpu.async_copy(h_hbm.at[src_v.at[0]], buf0, sem0)

            def body(i, carry):
                j = 2 * i
                pltpu.async_copy(h_hbm.at[src_v.at[j + 1]], buf1, sem1)
                pltpu.make_async_copy(h_hbm.at[src_v.at[j]], buf0, sem0).wait()
                pltpu.sync_copy(buf0, acc.at[dst_v.at[j]], add=True)
                pltpu.async_copy(h_hbm.at[src_v.at[j + 2]], buf0, sem0)
                pltpu.make_async_copy(
                    h_hbm.at[src_v.at[j + 1]], buf1, sem1).wait()
                pltpu.sync_copy(buf1, acc.at[dst_v.at[j + 1]], add=True)
                return carry

            lax.fori_loop(0, (CHH - 1) // 2, body, 0)
            pltpu.make_async_copy(
                h_hbm.at[src_v.at[CHH - 1]], buf0, sem0).wait()
            pltpu.sync_copy(buf0, acc.at[dst_v.at[CHH - 1]], add=True)

        plsc.subcore_barrier()
        pltpu.sync_copy(acc.at[pl.ds(sid * RPS, RPS)],
                        out_hbm.at[cid, pl.ds(sid * RPS, RPS)])

    return agg(h, src_r, dst_r, zrows)


BR = 1000


def _mlp_body(x_ref, a_ref, s_ref, w1_ref, b1_ref, w2_ref, b2_ref, o_ref):
    z = x_ref[...] * s_ref[...] + a_ref[0] + a_ref[1]
    z = jnp.maximum(
        jnp.dot(z, w1_ref[...], preferred_element_type=jnp.float32)
        + b1_ref[...], 0.0)
    z = jnp.maximum(
        jnp.dot(z, w2_ref[...], preferred_element_type=jnp.float32)
        + b2_ref[...], 0.0)
    o_ref[...] = z


def _mlp_tc(x, a, s, W1, b1, W2, b2):
    grid = (N // BR,)
    return pl.pallas_call(
        _mlp_body,
        grid=grid,
        in_specs=[
            pl.BlockSpec((BR, D), lambda i: (i, 0)),
            pl.BlockSpec((2, BR, D), lambda i: (0, i, 0)),
            pl.BlockSpec((1, D), lambda i: (0, 0)),
            pl.BlockSpec((D, H), lambda i: (0, 0)),
            pl.BlockSpec((1, H), lambda i: (0, 0)),
            pl.BlockSpec((H, H), lambda i: (0, 0)),
            pl.BlockSpec((1, H), lambda i: (0, 0)),
        ],
        out_specs=pl.BlockSpec((BR, H), lambda i: (i, 0)),
        out_shape=jax.ShapeDtypeStruct((N, H), jnp.float32),
    )(x, a, s, W1, b1, W2, b2)


def _mlp_pool_body(x_ref, a_ref, s_ref, w1_ref, b1_ref, w2_ref, b2_ref,
                   batch_ref, wc_ref, bc_ref, o_ref, acc_s, acc_c):
    i = pl.program_id(0)
    z = x_ref[...] * s_ref[...] + a_ref[0] + a_ref[1]
    z = jnp.maximum(
        jnp.dot(z, w1_ref[...], preferred_element_type=jnp.float32)
        + b1_ref[...], 0.0)
    h2 = jnp.maximum(
        jnp.dot(z, w2_ref[...], preferred_element_type=jnp.float32)
        + b2_ref[...], 0.0)
    b = batch_ref[0]
    gid = lax.broadcasted_iota(jnp.int32, (G, BR), 0)
    p = (gid == jnp.broadcast_to(b, (G, BR))).astype(jnp.float32)

    @pl.when(i == 0)
    def _():
        acc_s[...] = jnp.zeros_like(acc_s)
        acc_c[...] = jnp.zeros_like(acc_c)

    acc_s[...] += jnp.dot(p, h2, preferred_element_type=jnp.float32)
    acc_c[...] += jnp.broadcast_to(
        jnp.sum(p, axis=1, keepdims=True), (G, H))

    @pl.when(i == pl.num_programs(0) - 1)
    def _():
        rep = acc_s[...] / jnp.maximum(acc_c[...], 1.0)
        o_ref[...] = (
            jnp.dot(rep, wc_ref[...], preferred_element_type=jnp.float32)
            + bc_ref[...])


def _mlp_pool_tc(x, a, s, W1, b1, W2, b2, batch_r, Wc_pad, bc_pad):
    grid = (N // BR,)
    return pl.pallas_call(
        _mlp_pool_body,
        grid=grid,
        in_specs=[
            pl.BlockSpec((BR, D), lambda i: (i, 0)),
            pl.BlockSpec((2, BR, D), lambda i: (0, i, 0)),
            pl.BlockSpec((1, D), lambda i: (0, 0)),
            pl.BlockSpec((D, H), lambda i: (0, 0)),
            pl.BlockSpec((1, H), lambda i: (0, 0)),
            pl.BlockSpec((H, H), lambda i: (0, 0)),
            pl.BlockSpec((1, H), lambda i: (0, 0)),
            pl.BlockSpec((1, 1, BR), lambda i: (i, 0, 0)),
            pl.BlockSpec((H, 128), lambda i: (0, 0)),
            pl.BlockSpec((1, 128), lambda i: (0, 0)),
        ],
        out_specs=pl.BlockSpec((G, 128), lambda i: (0, 0)),
        out_shape=jax.ShapeDtypeStruct((G, 128), jnp.float32),
        scratch_shapes=[
            pltpu.VMEM((G, H), jnp.float32),
            pltpu.VMEM((G, H), jnp.float32),
        ],
    )(x, a, s, W1, b1, W2, b2, batch_r, Wc_pad, bc_pad)


def kernel(x, edge_index, batch, eps0, W1_0, b1_0, W2_0, b2_0,
           eps1, W1_1, b1_1, W2_1, b2_1, Wc, bc):
    src_r = edge_index[0].reshape(NW, 2, CHH, CH)
    dst_r = edge_index[1].reshape(NW, 2, CHH, CH)
    zrows = jnp.zeros((RPS, D), jnp.float32)
    ones_row = jnp.ones((1, D), jnp.float32)
    s0 = ones_row * (1.0 + eps0)
    s1 = ones_row * (1.0 + eps1)
    batch_r = batch.reshape(N // BR, 1, BR)
    Wc_pad = jnp.zeros((H, 128), jnp.float32).at[:, :C].set(Wc)
    bc_pad = jnp.zeros((1, 128), jnp.float32).at[0, :C].set(bc)

    a0 = _segment_sum_sc(x, src_r, dst_r, zrows)
    h1 = _mlp_tc(x, a0, s0, W1_0, b1_0.reshape(1, H), W2_0, b2_0.reshape(1, H))
    a1 = _segment_sum_sc(h1, src_r, dst_r, zrows)
    out = _mlp_pool_tc(h1, a1, s1, W1_1, b1_1.reshape(1, H),
                       W2_1, b2_1.reshape(1, H), batch_r, Wc_pad, bc_pad)
    return out[:, :C]
